# Initial kernel scaffold; baseline (speedup 1.0000x reference)
import jax
import jax.numpy as jnp
from jax.experimental import pallas as pl


def kernel(x_St, x_Sp, x_Stpm, x_Stpp, edges_fg, edges_bg, edges_gg, query, W_proj, W_fg, al_fg, ar_fg, W_bg, al_bg, ar_bg, W_gg, al_gg, ar_gg, Wq, Wk, W_out, b_out):
    raise NotImplementedError("write your pallas kernel here")


# speedup vs baseline: 28.6917x; 28.6917x over previous
import functools

import jax
import jax.numpy as jnp
from jax import lax
from jax.experimental import pallas as pl
from jax.experimental.pallas import tpu as pltpu
from jax.experimental.pallas import tpu_sc as plsc

N = 10000
E = 320000
FEAT = 128
H = 128
EMB = 4 * H
QD = 32
NL = 12

NC = 2
NS = 16
NW = NC * NS
EPW = E // NW
B = 80
NB = EPW // B
GPB = B // 16
DR = 640

R = 400
GRID = N // R


def _s1_body(x_ref, wp_ref, w3_ref, al_ref, ar_ref,
             z_ref, h0_ref, el_ref, er_ref):
    wp = wp_ref[...]
    for xi in range(4):
        xb = x_ref[xi]
        h0_ref[xi] = jnp.dot(xb, wp, preferred_element_type=jnp.float32)
        for g in range(3):
            z = jnp.dot(xb, w3_ref[g], preferred_element_type=jnp.float32)
            l = g * 4 + xi
            z_ref[l] = z
            el_ref[l, 0, 0] = jnp.sum(z * al_ref[g, 0][None, :], axis=1)
            er_ref[l, 0, 0] = jnp.sum(z * ar_ref[g, 0][None, :], axis=1)


def _stage1(X, W_proj, W3, al3, ar3):
    return pl.pallas_call(
        _s1_body,
        grid=(GRID,),
        in_specs=[
            pl.BlockSpec((4, R, FEAT), lambda i: (0, i, 0)),
            pl.BlockSpec((FEAT, H), lambda i: (0, 0)),
            pl.BlockSpec((3, FEAT, H), lambda i: (0, 0, 0)),
            pl.BlockSpec((3, 1, H), lambda i: (0, 0, 0)),
            pl.BlockSpec((3, 1, H), lambda i: (0, 0, 0)),
        ],
        out_specs=[
            pl.BlockSpec((NL, R, H), lambda i: (0, i, 0)),
            pl.BlockSpec((4, R, H), lambda i: (0, i, 0)),
            pl.BlockSpec((NL, 1, 1, R), lambda i: (0, i, 0, 0)),
            pl.BlockSpec((NL, 1, 1, R), lambda i: (0, i, 0, 0)),
        ],
        out_shape=[
            jax.ShapeDtypeStruct((NL, N, H), jnp.float32),
            jax.ShapeDtypeStruct((4, N, H), jnp.float32),
            jax.ShapeDtypeStruct((NL, GRID, 1, R), jnp.float32),
            jax.ShapeDtypeStruct((NL, GRID, 1, R), jnp.float32),
        ],
    )(X, W_proj, W3, al3, ar3)


EC = 2000
NCH = EPW // EC
BPC = EC // B


def _sc_body(z_hbm, el_hbm, er_hbm, efg, ebg, egg,
             num_hbm, den_hbm,
             el_v, er_v, den_v, rows_v, srcc_v, dstc_v,
             ee80, gidx80, dsti80, ident2, acc_sh, den_sh, sem):
    cid = lax.axis_index("c")
    sid = lax.axis_index("s")
    wid = cid * NS + sid

    z16f = jnp.zeros((16,), jnp.float32)
    iota16 = lax.iota(jnp.int32, 16)
    for k in range(5):
        for j in range(8):
            ident2[k, pl.ds(j * 16, 16)] = iota16 + (k * 128 + j * 16)

    for g, e_ref in enumerate((efg, ebg, egg)):

        def _layer(xi, _, g=g, e_ref=e_ref):
            l = g * 4 + xi
            plsc.subcore_barrier()
            def _zr(k, _):
                for h8 in range(8):
                    rows_v[k, pl.ds(h8 * 16, 16)] = z16f
                return 0
            lax.fori_loop(0, B, _zr, 0)
            def _zd(k, _):
                den_v[k] = z16f
                return 0
            lax.fori_loop(0, DR, _zd, 0)
            for j in range(7):
                pltpu.sync_copy(rows_v,
                                acc_sh.at[pl.ds(sid * 625 + j * 80, 80)])
            pltpu.sync_copy(rows_v.at[pl.ds(0, 65)],
                            acc_sh.at[pl.ds(sid * 625 + 560, 65)])
            pltpu.sync_copy(den_v.at[pl.ds(0, 40)],
                            den_sh.at[pl.ds(sid * 40, 40)])
            pltpu.sync_copy(el_hbm.at[l], el_v)
            pltpu.sync_copy(er_hbm.at[l], er_v)
            plsc.subcore_barrier()

            lN = l * N

            for c5 in range(NCH):
                base = wid * EPW + c5 * EC
                pltpu.sync_copy(e_ref.at[pl.ds(base, EC)], srcc_v)
                pltpu.sync_copy(e_ref.at[pl.ds(E + base, EC)], dstc_v)

                def _batch(b, _):
                    for j in range(GPB):
                        s16 = srcc_v[pl.ds(b * B + j * 16, 16)]
                        d16 = dstc_v[pl.ds(b * B + j * 16, 16)]
                        a = (plsc.load_gather(el_v, [s16])
                             + plsc.load_gather(er_v, [d16]))
                        e = jnp.where(a >= 0.0, a, a * jnp.float32(0.2))
                        ee = jnp.exp(e)
                        ee80[pl.ds(j * 16, 16)] = ee
                        plsc.addupdate_scatter(den_v, [d16 >> 4, d16 & 15], ee)
                        gidx80[pl.ds(j * 16, 16)] = s16 + lN
                        dsti80[pl.ds(j * 16, 16)] = d16
                    pltpu.async_copy(z_hbm.at[gidx80], rows_v, sem).wait()
                    def _row(r, _):
                        sv = plsc.load_gather(
                            ee80, [jnp.broadcast_to(r, (16,))])
                        for h8 in range(8):
                            rows_v[r, pl.ds(h8 * 16, 16)] = (
                                rows_v[r, pl.ds(h8 * 16, 16)] * sv)
                        return 0
                    lax.fori_loop(0, B, _row, 0)
                    pltpu.sync_copy(rows_v, acc_sh.at[dsti80], add=True)
                    return 0
                lax.fori_loop(0, BPC, _batch, 0)

            for k in range(5):
                pltpu.sync_copy(den_v.at[pl.ds(k * 128, 128)],
                                den_sh.at[ident2.at[k]], add=True)

            plsc.subcore_barrier()
            @pl.when(sid == 0)
            def _dump():
                pltpu.sync_copy(acc_sh, num_hbm.at[l, cid])
                pltpu.sync_copy(den_sh, den_hbm.at[l, cid])
            return 0

        lax.fori_loop(0, 4, _layer, 0)


def _stage2(zf, el12, er12, efg, ebg, egg):
    mesh = plsc.VectorSubcoreMesh(core_axis_name="c", subcore_axis_name="s",
                                  num_cores=NC, num_subcores=NS)
    f = pl.kernel(
        _sc_body,
        out_type=(
            jax.ShapeDtypeStruct((NL, NC, N, H), jnp.float32),
            jax.ShapeDtypeStruct((NL, NC, DR, 16), jnp.float32),
        ),
        mesh=mesh,
        compiler_params=pltpu.CompilerParams(needs_layout_passes=False,
                                             use_tc_tiling_on_sc=False),
        scratch_types=[
            pltpu.VMEM((N,), jnp.float32),
            pltpu.VMEM((N,), jnp.float32),
            pltpu.VMEM((DR, 16), jnp.float32),
            pltpu.VMEM((B, H), jnp.float32),
            pltpu.VMEM((EC,), jnp.int32),
            pltpu.VMEM((EC,), jnp.int32),
            pltpu.VMEM((B,), jnp.float32),
            pltpu.VMEM((B,), jnp.int32),
            pltpu.VMEM((B,), jnp.int32),
            pltpu.VMEM((5, 128), jnp.int32),
            pltpu.VMEM_SHARED((N, H), jnp.float32),
            pltpu.VMEM_SHARED((DR, 16), jnp.float32),
            pltpu.SemaphoreType.DMA,
        ],
    )
    return f(zf, el12, er12, efg, ebg, egg)


def _s3_body(num_ref, den_ref, h0_ref, q_ref, wq_ref, wk_ref, wo_ref, b_ref,
             out_ref):
    wk = wk_ref[...]
    qp = jnp.dot(q_ref[...], wq_ref[...], preferred_element_type=jnp.float32)
    hs = []
    scores = []
    for xi in range(4):
        parts = [h0_ref[xi]]
        for g in range(3):
            l = g * 4 + xi
            nm = num_ref[l, 0] + num_ref[l, 1]
            dn = den_ref[l, 0, 0, 0] + den_ref[l, 1, 0, 0] + jnp.float32(1e-9)
            parts.append(nm / dn[:, None])
        h = jax.nn.relu(jnp.concatenate(parts, axis=1))
        hs.append(h)
        keys = jnp.dot(h, wk, preferred_element_type=jnp.float32)
        scores.append(jnp.sum(qp * keys, axis=1))
    s = jnp.stack(scores, axis=0) * jnp.float32(1.0 / (EMB ** 0.5))
    m = jnp.max(s, axis=0, keepdims=True)
    ex = jnp.exp(s - m)
    w = ex / jnp.sum(ex, axis=0, keepdims=True)
    temp = w[0][:, None] * hs[0]
    for t in range(1, 4):
        temp = temp + w[t][:, None] * hs[t]
    temp = jax.nn.relu(temp)
    out_ref[0, 0] = jnp.sum(temp * wo_ref[0][None, :], axis=1) + b_ref[0, 0]


def _stage3(num, den, h0, query, Wq, Wk, wo_row, b2):
    return pl.pallas_call(
        _s3_body,
        grid=(GRID,),
        in_specs=[
            pl.BlockSpec((NL, NC, R, H), lambda i: (0, 0, i, 0)),
            pl.BlockSpec((NL, NC, 1, 1, R), lambda i: (0, 0, i, 0, 0)),
            pl.BlockSpec((4, R, H), lambda i: (0, i, 0)),
            pl.BlockSpec((R, QD), lambda i: (i, 0)),
            pl.BlockSpec((QD, EMB), lambda i: (0, 0)),
            pl.BlockSpec((EMB, EMB), lambda i: (0, 0)),
            pl.BlockSpec((1, EMB), lambda i: (0, 0)),
            pl.BlockSpec((1, 1), lambda i: (0, 0)),
        ],
        out_specs=pl.BlockSpec((1, 1, R), lambda i: (i, 0, 0)),
        out_shape=jax.ShapeDtypeStruct((GRID, 1, R), jnp.float32),
    )(num, den, h0, query, Wq, Wk, wo_row, b2)


def kernel(x_St, x_Sp, x_Stpm, x_Stpp, edges_fg, edges_bg, edges_gg, query,
           W_proj, W_fg, al_fg, ar_fg, W_bg, al_bg, ar_bg, W_gg, al_gg, ar_gg,
           Wq, Wk, W_out, b_out):
    X = jnp.stack([x_St, x_Sp, x_Stpm, x_Stpp])
    W3 = jnp.stack([W_fg, W_bg, W_gg])
    al3 = jnp.stack([al_fg, al_bg, al_gg]).reshape(3, 1, H)
    ar3 = jnp.stack([ar_fg, ar_bg, ar_gg]).reshape(3, 1, H)
    Z, h0, el12, er12 = _stage1(X, W_proj, W3, al3, ar3)
    el12 = el12.reshape(NL, N)
    er12 = er12.reshape(NL, N)
    zf = Z.reshape(NL * N, H)
    num, den = _stage2(zf, el12, er12, edges_fg.reshape(2 * E),
                       edges_bg.reshape(2 * E), edges_gg.reshape(2 * E))
    den = den.reshape(NL, NC, DR * 16)[:, :, :N].reshape(NL, NC, GRID, 1, R)
    out = _stage3(num, den, h0, query, Wq, Wk,
                  W_out.reshape(1, EMB), b_out.reshape(1, 1))
    return out.reshape(N, 1)


# speedup vs baseline: 36.1667x; 1.2605x over previous
import functools

import jax
import jax.numpy as jnp
from jax import lax
from jax.experimental import pallas as pl
from jax.experimental.pallas import tpu as pltpu
from jax.experimental.pallas import tpu_sc as plsc

N = 10000
E = 320000
FEAT = 128
H = 128
EMB = 4 * H
QD = 32
NL = 12

NC = 2
NS = 16
NW = NC * NS
EPW = E // NW
B = 80
NB = EPW // B
GPB = B // 16
DR = 640

R = 400
GRID = N // R


def _s1_body(x_ref, wp_ref, w3_ref, al_ref, ar_ref,
             z_ref, h0_ref, el_ref, er_ref):
    wp = wp_ref[...]
    for xi in range(4):
        xb = x_ref[xi]
        h0_ref[xi] = jnp.dot(xb, wp, preferred_element_type=jnp.float32)
        for g in range(3):
            z = jnp.dot(xb, w3_ref[g], preferred_element_type=jnp.float32)
            l = g * 4 + xi
            z_ref[0, l] = z[:, 0:64]
            z_ref[1, l] = z[:, 64:128]
            el_ref[l, 0, 0] = jnp.sum(z * al_ref[g, 0][None, :], axis=1)
            er_ref[l, 0, 0] = jnp.sum(z * ar_ref[g, 0][None, :], axis=1)


def _stage1(X, W_proj, W3, al3, ar3):
    return pl.pallas_call(
        _s1_body,
        grid=(GRID,),
        in_specs=[
            pl.BlockSpec((4, R, FEAT), lambda i: (0, i, 0)),
            pl.BlockSpec((FEAT, H), lambda i: (0, 0)),
            pl.BlockSpec((3, FEAT, H), lambda i: (0, 0, 0)),
            pl.BlockSpec((3, 1, H), lambda i: (0, 0, 0)),
            pl.BlockSpec((3, 1, H), lambda i: (0, 0, 0)),
        ],
        out_specs=[
            pl.BlockSpec((2, NL, R, 64), lambda i: (0, 0, i, 0)),
            pl.BlockSpec((4, R, H), lambda i: (0, i, 0)),
            pl.BlockSpec((NL, 1, 1, R), lambda i: (0, i, 0, 0)),
            pl.BlockSpec((NL, 1, 1, R), lambda i: (0, i, 0, 0)),
        ],
        out_shape=[
            jax.ShapeDtypeStruct((2, NL, N, 64), jnp.float32),
            jax.ShapeDtypeStruct((4, N, H), jnp.float32),
            jax.ShapeDtypeStruct((NL, GRID, 1, R), jnp.float32),
            jax.ShapeDtypeStruct((NL, GRID, 1, R), jnp.float32),
        ],
    )(X, W_proj, W3, al3, ar3)


NU = 5
NP = NB // NU
HW = 64


def _sc_body(z_hbm, el_hbm, er_hbm, efg, ebg, egg,
             num_hbm, den_hbm,
             el_v, er_v, den_v, src_v, dst_v,
             r0, r1, r2, r3, r4,
             ee0, ee1, ee2, ee3, ee4,
             gi0, gi1, gi2, gi3, gi4,
             di0, di1, di2, di3, di4,
             ident2, acc_sh, den_sh, gsem, ssem):
    cid = lax.axis_index("c")
    sid = lax.axis_index("s")
    wid = cid * NS + sid
    rows = (r0, r1, r2, r3, r4)
    ees = (ee0, ee1, ee2, ee3, ee4)
    gis = (gi0, gi1, gi2, gi3, gi4)
    dis = (di0, di1, di2, di3, di4)

    z16f = jnp.zeros((16,), jnp.float32)
    iota16 = lax.iota(jnp.int32, 16)
    for k in range(5):
        for j in range(8):
            ident2[k, pl.ds(j * 16, 16)] = iota16 + (k * 128 + j * 16)

    for g, e_ref in enumerate((efg, ebg, egg)):
        pltpu.sync_copy(e_ref.at[pl.ds(wid * EPW, EPW)], src_v)
        pltpu.sync_copy(e_ref.at[pl.ds(E + wid * EPW, EPW)], dst_v)

        def _layer(xi, _, g=g):
            l = g * 4 + xi
            plsc.subcore_barrier()

            def _zero_acc():
                def _zr(k, _):
                    for h4 in range(HW // 16):
                        r0[k, pl.ds(h4 * 16, 16)] = z16f
                    return 0
                lax.fori_loop(0, B, _zr, 0)
                for j in range(7):
                    pltpu.sync_copy(r0, acc_sh.at[pl.ds(sid * 625 + j * 80, 80)])
                pltpu.sync_copy(r0.at[pl.ds(0, 65)],
                                acc_sh.at[pl.ds(sid * 625 + 560, 65)])

            _zero_acc()
            def _zd(k, _):
                den_v[k] = z16f
                return 0
            lax.fori_loop(0, DR, _zd, 0)
            pltpu.sync_copy(den_v.at[pl.ds(0, 40)],
                            den_sh.at[pl.ds(sid * 40, 40)])
            pltpu.sync_copy(el_hbm.at[l], el_v)
            pltpu.sync_copy(er_hbm.at[l], er_v)
            plsc.subcore_barrier()

            for h in range(2):
                off = h * NL * N + l * N

                def _step(p, _, h=h):
                    for u in range(NU):
                        b = p * NU + u
                        @pl.when(p > 0)
                        def _ws(u=u, b=b):
                            pltpu.make_async_copy(
                                rows[u], acc_sh.at[dis[u]], ssem).wait()
                        for j in range(GPB):
                            s16 = src_v[pl.ds(b * B + j * 16, 16)]
                            d16 = dst_v[pl.ds(b * B + j * 16, 16)]
                            a = (plsc.load_gather(el_v, [s16])
                                 + plsc.load_gather(er_v, [d16]))
                            e = jnp.where(a >= 0.0, a, a * jnp.float32(0.2))
                            ee = jnp.exp(e)
                            ees[u][pl.ds(j * 16, 16)] = ee
                            if h == 0:
                                plsc.addupdate_scatter(
                                    den_v, [d16 >> 4, d16 & 15], ee)
                            gis[u][pl.ds(j * 16, 16)] = s16 + off
                            dis[u][pl.ds(j * 16, 16)] = d16
                        pltpu.async_copy(z_hbm.at[gis[u]], rows[u], gsem)
                    for u in range(NU):
                        pltpu.make_async_copy(
                            z_hbm.at[gis[u]], rows[u], gsem).wait()
                        def _row(r, _, u=u):
                            sv = plsc.load_gather(
                                ees[u], [jnp.broadcast_to(r, (16,))])
                            for h4 in range(HW // 16):
                                rows[u][r, pl.ds(h4 * 16, 16)] = (
                                    rows[u][r, pl.ds(h4 * 16, 16)] * sv)
                            return 0
                        lax.fori_loop(0, B, _row, 0)
                        pltpu.async_copy(rows[u], acc_sh.at[dis[u]], ssem,
                                         add=True)
                    return 0
                lax.fori_loop(0, NP, _step, 0)
                for u in range(NU):
                    pltpu.make_async_copy(rows[u], acc_sh.at[dis[u]],
                                          ssem).wait()

                if h == 0:
                    for k in range(5):
                        pltpu.sync_copy(den_v.at[pl.ds(k * 128, 128)],
                                        den_sh.at[ident2.at[k]], add=True)

                plsc.subcore_barrier()
                @pl.when(sid == 0)
                def _dump(h=h):
                    pltpu.sync_copy(acc_sh, num_hbm.at[l, cid, h])
                    if h == 0:
                        pltpu.sync_copy(den_sh, den_hbm.at[l, cid])
                if h == 0:
                    plsc.subcore_barrier()
                    _zero_acc()
                    plsc.subcore_barrier()
            return 0

        lax.fori_loop(0, 4, _layer, 0)


def _stage2(zf, el12, er12, efg, ebg, egg):
    mesh = plsc.VectorSubcoreMesh(core_axis_name="c", subcore_axis_name="s",
                                  num_cores=NC, num_subcores=NS)
    f = pl.kernel(
        _sc_body,
        out_type=(
            jax.ShapeDtypeStruct((NL, NC, 2, N, HW), jnp.float32),
            jax.ShapeDtypeStruct((NL, NC, DR, 16), jnp.float32),
        ),
        mesh=mesh,
        compiler_params=pltpu.CompilerParams(needs_layout_passes=False,
                                             use_tc_tiling_on_sc=False),
        scratch_types=(
            [pltpu.VMEM((N,), jnp.float32),
             pltpu.VMEM((N,), jnp.float32),
             pltpu.VMEM((DR, 16), jnp.float32),
             pltpu.VMEM((EPW,), jnp.int32),
             pltpu.VMEM((EPW,), jnp.int32)]
            + [pltpu.VMEM((B, HW), jnp.float32)] * NU
            + [pltpu.VMEM((B,), jnp.float32)] * NU
            + [pltpu.VMEM((B,), jnp.int32)] * NU
            + [pltpu.VMEM((B,), jnp.int32)] * NU
            + [pltpu.VMEM((5, 128), jnp.int32),
               pltpu.VMEM_SHARED((N, HW), jnp.float32),
               pltpu.VMEM_SHARED((DR, 16), jnp.float32),
               pltpu.SemaphoreType.DMA,
               pltpu.SemaphoreType.DMA]
        ),
    )
    return f(zf, el12, er12, efg, ebg, egg)


def _s3_body(num_ref, den_ref, h0_ref, q_ref, wq_ref, wk_ref, wo_ref, b_ref,
             out_ref):
    wk = wk_ref[...]
    qp = jnp.dot(q_ref[...], wq_ref[...], preferred_element_type=jnp.float32)
    hs = []
    scores = []
    for xi in range(4):
        parts = [h0_ref[xi]]
        for g in range(3):
            l = g * 4 + xi
            nm = jnp.concatenate(
                [num_ref[l, 0, 0] + num_ref[l, 1, 0],
                 num_ref[l, 0, 1] + num_ref[l, 1, 1]], axis=1)
            dn = den_ref[l, 0, 0, 0] + den_ref[l, 1, 0, 0] + jnp.float32(1e-9)
            parts.append(nm / dn[:, None])
        h = jax.nn.relu(jnp.concatenate(parts, axis=1))
        hs.append(h)
        keys = jnp.dot(h, wk, preferred_element_type=jnp.float32)
        scores.append(jnp.sum(qp * keys, axis=1))
    s = jnp.stack(scores, axis=0) * jnp.float32(1.0 / (EMB ** 0.5))
    m = jnp.max(s, axis=0, keepdims=True)
    ex = jnp.exp(s - m)
    w = ex / jnp.sum(ex, axis=0, keepdims=True)
    temp = w[0][:, None] * hs[0]
    for t in range(1, 4):
        temp = temp + w[t][:, None] * hs[t]
    temp = jax.nn.relu(temp)
    out_ref[0, 0] = jnp.sum(temp * wo_ref[0][None, :], axis=1) + b_ref[0, 0]


def _stage3(num, den, h0, query, Wq, Wk, wo_row, b2):
    return pl.pallas_call(
        _s3_body,
        grid=(GRID,),
        in_specs=[
            pl.BlockSpec((NL, NC, 2, R, HW), lambda i: (0, 0, 0, i, 0)),
            pl.BlockSpec((NL, NC, 1, 1, R), lambda i: (0, 0, i, 0, 0)),
            pl.BlockSpec((4, R, H), lambda i: (0, i, 0)),
            pl.BlockSpec((R, QD), lambda i: (i, 0)),
            pl.BlockSpec((QD, EMB), lambda i: (0, 0)),
            pl.BlockSpec((EMB, EMB), lambda i: (0, 0)),
            pl.BlockSpec((1, EMB), lambda i: (0, 0)),
            pl.BlockSpec((1, 1), lambda i: (0, 0)),
        ],
        out_specs=pl.BlockSpec((1, 1, R), lambda i: (i, 0, 0)),
        out_shape=jax.ShapeDtypeStruct((GRID, 1, R), jnp.float32),
    )(num, den, h0, query, Wq, Wk, wo_row, b2)


def kernel(x_St, x_Sp, x_Stpm, x_Stpp, edges_fg, edges_bg, edges_gg, query,
           W_proj, W_fg, al_fg, ar_fg, W_bg, al_bg, ar_bg, W_gg, al_gg, ar_gg,
           Wq, Wk, W_out, b_out):
    X = jnp.stack([x_St, x_Sp, x_Stpm, x_Stpp])
    W3 = jnp.stack([W_fg, W_bg, W_gg])
    al3 = jnp.stack([al_fg, al_bg, al_gg]).reshape(3, 1, H)
    ar3 = jnp.stack([ar_fg, ar_bg, ar_gg]).reshape(3, 1, H)
    Z, h0, el12, er12 = _stage1(X, W_proj, W3, al3, ar3)
    el12 = el12.reshape(NL, N)
    er12 = er12.reshape(NL, N)
    zf = Z.reshape(2 * NL * N, HW)
    num, den = _stage2(zf, el12, er12, edges_fg.reshape(2 * E),
                       edges_bg.reshape(2 * E), edges_gg.reshape(2 * E))
    den = den.reshape(NL, NC, DR * 16)[:, :, :N].reshape(NL, NC, GRID, 1, R)
    out = _stage3(num, den, h0, query, Wq, Wk,
                  W_out.reshape(1, EMB), b_out.reshape(1, 1))
    return out.reshape(N, 1)


# speedup vs baseline: 38.3835x; 1.0613x over previous
import functools

import jax
import jax.numpy as jnp
from jax import lax
from jax.experimental import pallas as pl
from jax.experimental.pallas import tpu as pltpu
from jax.experimental.pallas import tpu_sc as plsc

N = 10000
E = 320000
FEAT = 128
H = 128
EMB = 4 * H
QD = 32
NL = 12

NC = 2
NS = 16
NW = NC * NS
EPW = E // NW
B = 80
NB = EPW // B
GPB = B // 16
DR = 640

R = 400
GRID = N // R


def _s1_body(x_ref, wp_ref, w3_ref, al_ref, ar_ref,
             z_ref, h0_ref, el_ref, er_ref):
    wp = wp_ref[...]
    for xi in range(4):
        xb = x_ref[xi]
        h0_ref[xi] = jnp.dot(xb, wp, preferred_element_type=jnp.float32)
        for g in range(3):
            z = jnp.dot(xb, w3_ref[g], preferred_element_type=jnp.float32)
            l = g * 4 + xi
            z_ref[0, l] = z[:, 0:64]
            z_ref[1, l] = z[:, 64:128]
            el_ref[l, 0, 0] = jnp.sum(z * al_ref[g, 0][None, :], axis=1)
            er_ref[l, 0, 0] = jnp.sum(z * ar_ref[g, 0][None, :], axis=1)


def _stage1(X, W_proj, W3, al3, ar3):
    return pl.pallas_call(
        _s1_body,
        grid=(GRID,),
        in_specs=[
            pl.BlockSpec((4, R, FEAT), lambda i: (0, i, 0)),
            pl.BlockSpec((FEAT, H), lambda i: (0, 0)),
            pl.BlockSpec((3, FEAT, H), lambda i: (0, 0, 0)),
            pl.BlockSpec((3, 1, H), lambda i: (0, 0, 0)),
            pl.BlockSpec((3, 1, H), lambda i: (0, 0, 0)),
        ],
        out_specs=[
            pl.BlockSpec((2, NL, R, 64), lambda i: (0, 0, i, 0)),
            pl.BlockSpec((4, R, H), lambda i: (0, i, 0)),
            pl.BlockSpec((NL, 1, 1, R), lambda i: (0, i, 0, 0)),
            pl.BlockSpec((NL, 1, 1, R), lambda i: (0, i, 0, 0)),
        ],
        out_shape=[
            jax.ShapeDtypeStruct((2, NL, N, 64), jnp.float32),
            jax.ShapeDtypeStruct((4, N, H), jnp.float32),
            jax.ShapeDtypeStruct((NL, GRID, 1, R), jnp.float32),
            jax.ShapeDtypeStruct((NL, GRID, 1, R), jnp.float32),
        ],
    )(X, W_proj, W3, al3, ar3)


NU = 5
NP = NB // NU
HW = 64


def _sc_body(z_hbm, el_hbm, er_hbm, efg, ebg, egg,
             num_hbm, den_hbm,
             el_v, er_v, den_v, src_v, dst_v,
             r0, r1, r2, r3, r4,
             ee0, ee1, ee2, ee3, ee4,
             gi0, gi1, gi2, gi3, gi4,
             di0, di1, di2, di3, di4,
             ident2, acc_sh, den_sh, gsem, ssem):
    cid = lax.axis_index("c")
    sid = lax.axis_index("s")
    wid = cid * NS + sid
    rows = (r0, r1, r2, r3, r4)
    ees = (ee0, ee1, ee2, ee3, ee4)
    gis = (gi0, gi1, gi2, gi3, gi4)
    dis = (di0, di1, di2, di3, di4)

    z16f = jnp.zeros((16,), jnp.float32)
    iota16 = lax.iota(jnp.int32, 16)
    for k in range(5):
        for j in range(8):
            ident2[k, pl.ds(j * 16, 16)] = iota16 + (k * 128 + j * 16)

    for g, e_ref in enumerate((efg, ebg, egg)):
        pltpu.sync_copy(e_ref.at[pl.ds(wid * EPW, EPW)], src_v)
        pltpu.sync_copy(e_ref.at[pl.ds(E + wid * EPW, EPW)], dst_v)

        def _layer(xi, _, g=g):
            l = g * 4 + xi
            plsc.subcore_barrier()

            def _zero_acc():
                def _zr(k, _):
                    for h4 in range(HW // 16):
                        r0[k, pl.ds(h4 * 16, 16)] = z16f
                    return 0
                lax.fori_loop(0, B, _zr, 0)
                for j in range(7):
                    pltpu.sync_copy(r0, acc_sh.at[pl.ds(sid * 625 + j * 80, 80)])
                pltpu.sync_copy(r0.at[pl.ds(0, 65)],
                                acc_sh.at[pl.ds(sid * 625 + 560, 65)])

            _zero_acc()
            def _zd(k, _):
                den_v[k] = z16f
                return 0
            lax.fori_loop(0, DR, _zd, 0)
            pltpu.sync_copy(den_v.at[pl.ds(0, 40)],
                            den_sh.at[pl.ds(sid * 40, 40)])
            pltpu.sync_copy(el_hbm.at[l], el_v)
            pltpu.sync_copy(er_hbm.at[l], er_v)
            plsc.subcore_barrier()

            for h in range(2):
                off = h * NL * N + l * N

                def _step(p, _, h=h):
                    for u in range(NU):
                        b = p * NU + u
                        @pl.when(p > 0)
                        def _ws(u=u, b=b):
                            pltpu.make_async_copy(
                                rows[u], acc_sh.at[dis[u]], ssem).wait()
                        for j in range(GPB):
                            s16 = src_v[pl.ds(b * B + j * 16, 16)]
                            d16 = dst_v[pl.ds(b * B + j * 16, 16)]
                            a = (plsc.load_gather(el_v, [s16])
                                 + plsc.load_gather(er_v, [d16]))
                            e = jnp.where(a >= 0.0, a, a * jnp.float32(0.2))
                            ee = jnp.exp(e)
                            ees[u][pl.ds(j * 16, 16)] = ee
                            if h == 0:
                                plsc.addupdate_scatter(
                                    den_v, [d16 >> 4, d16 & 15], ee)
                            gis[u][pl.ds(j * 16, 16)] = s16 + off
                            dis[u][pl.ds(j * 16, 16)] = d16
                        pltpu.async_copy(z_hbm.at[gis[u]], rows[u], gsem)
                    for u in range(NU):
                        pltpu.make_async_copy(
                            z_hbm.at[gis[u]], rows[u], gsem).wait()
                        def _row(r5, _, u=u):
                            for q in range(5):
                                r = r5 * 5 + q
                                sv = plsc.load_gather(
                                    ees[u], [jnp.broadcast_to(r, (16,))])
                                for h4 in range(HW // 16):
                                    rows[u][r, pl.ds(h4 * 16, 16)] = (
                                        rows[u][r, pl.ds(h4 * 16, 16)] * sv)
                            return 0
                        lax.fori_loop(0, B // 5, _row, 0)
                        pltpu.async_copy(rows[u], acc_sh.at[dis[u]], ssem,
                                         add=True)
                    return 0
                lax.fori_loop(0, NP, _step, 0)
                for u in range(NU):
                    pltpu.make_async_copy(rows[u], acc_sh.at[dis[u]],
                                          ssem).wait()

                if h == 0:
                    for k in range(5):
                        pltpu.sync_copy(den_v.at[pl.ds(k * 128, 128)],
                                        den_sh.at[ident2.at[k]], add=True)

                plsc.subcore_barrier()
                @pl.when(sid == 0)
                def _dump(h=h):
                    pltpu.sync_copy(acc_sh, num_hbm.at[l, cid, h])
                    if h == 0:
                        pltpu.sync_copy(den_sh, den_hbm.at[l, cid])
                if h == 0:
                    plsc.subcore_barrier()
                    _zero_acc()
                    plsc.subcore_barrier()
            return 0

        lax.fori_loop(0, 4, _layer, 0)


def _stage2(zf, el12, er12, efg, ebg, egg):
    mesh = plsc.VectorSubcoreMesh(core_axis_name="c", subcore_axis_name="s",
                                  num_cores=NC, num_subcores=NS)
    f = pl.kernel(
        _sc_body,
        out_type=(
            jax.ShapeDtypeStruct((NL, NC, 2, N, HW), jnp.float32),
            jax.ShapeDtypeStruct((NL, NC, DR, 16), jnp.float32),
        ),
        mesh=mesh,
        compiler_params=pltpu.CompilerParams(needs_layout_passes=False,
                                             use_tc_tiling_on_sc=False),
        scratch_types=(
            [pltpu.VMEM((N,), jnp.float32),
             pltpu.VMEM((N,), jnp.float32),
             pltpu.VMEM((DR, 16), jnp.float32),
             pltpu.VMEM((EPW,), jnp.int32),
             pltpu.VMEM((EPW,), jnp.int32)]
            + [pltpu.VMEM((B, HW), jnp.float32)] * NU
            + [pltpu.VMEM((B,), jnp.float32)] * NU
            + [pltpu.VMEM((B,), jnp.int32)] * NU
            + [pltpu.VMEM((B,), jnp.int32)] * NU
            + [pltpu.VMEM((5, 128), jnp.int32),
               pltpu.VMEM_SHARED((N, HW), jnp.float32),
               pltpu.VMEM_SHARED((DR, 16), jnp.float32),
               pltpu.SemaphoreType.DMA,
               pltpu.SemaphoreType.DMA]
        ),
    )
    return f(zf, el12, er12, efg, ebg, egg)


def _s3_body(num_ref, den_ref, h0_ref, q_ref, wq_ref, wk_ref, wo_ref, b_ref,
             out_ref):
    wk = wk_ref[...]
    qp = jnp.dot(q_ref[...], wq_ref[...], preferred_element_type=jnp.float32)
    hs = []
    scores = []
    for xi in range(4):
        parts = [h0_ref[xi]]
        for g in range(3):
            l = g * 4 + xi
            nm = jnp.concatenate(
                [num_ref[l, 0, 0] + num_ref[l, 1, 0],
                 num_ref[l, 0, 1] + num_ref[l, 1, 1]], axis=1)
            dn = den_ref[l, 0, 0, 0] + den_ref[l, 1, 0, 0] + jnp.float32(1e-9)
            parts.append(nm / dn[:, None])
        h = jax.nn.relu(jnp.concatenate(parts, axis=1))
        hs.append(h)
        keys = jnp.dot(h, wk, preferred_element_type=jnp.float32)
        scores.append(jnp.sum(qp * keys, axis=1))
    s = jnp.stack(scores, axis=0) * jnp.float32(1.0 / (EMB ** 0.5))
    m = jnp.max(s, axis=0, keepdims=True)
    ex = jnp.exp(s - m)
    w = ex / jnp.sum(ex, axis=0, keepdims=True)
    temp = w[0][:, None] * hs[0]
    for t in range(1, 4):
        temp = temp + w[t][:, None] * hs[t]
    temp = jax.nn.relu(temp)
    out_ref[0, 0] = jnp.sum(temp * wo_ref[0][None, :], axis=1) + b_ref[0, 0]


def _stage3(num, den, h0, query, Wq, Wk, wo_row, b2):
    return pl.pallas_call(
        _s3_body,
        grid=(GRID,),
        in_specs=[
            pl.BlockSpec((NL, NC, 2, R, HW), lambda i: (0, 0, 0, i, 0)),
            pl.BlockSpec((NL, NC, 1, 1, R), lambda i: (0, 0, i, 0, 0)),
            pl.BlockSpec((4, R, H), lambda i: (0, i, 0)),
            pl.BlockSpec((R, QD), lambda i: (i, 0)),
            pl.BlockSpec((QD, EMB), lambda i: (0, 0)),
            pl.BlockSpec((EMB, EMB), lambda i: (0, 0)),
            pl.BlockSpec((1, EMB), lambda i: (0, 0)),
            pl.BlockSpec((1, 1), lambda i: (0, 0)),
        ],
        out_specs=pl.BlockSpec((1, 1, R), lambda i: (i, 0, 0)),
        out_shape=jax.ShapeDtypeStruct((GRID, 1, R), jnp.float32),
    )(num, den, h0, query, Wq, Wk, wo_row, b2)


def kernel(x_St, x_Sp, x_Stpm, x_Stpp, edges_fg, edges_bg, edges_gg, query,
           W_proj, W_fg, al_fg, ar_fg, W_bg, al_bg, ar_bg, W_gg, al_gg, ar_gg,
           Wq, Wk, W_out, b_out):
    X = jnp.stack([x_St, x_Sp, x_Stpm, x_Stpp])
    W3 = jnp.stack([W_fg, W_bg, W_gg])
    al3 = jnp.stack([al_fg, al_bg, al_gg]).reshape(3, 1, H)
    ar3 = jnp.stack([ar_fg, ar_bg, ar_gg]).reshape(3, 1, H)
    Z, h0, el12, er12 = _stage1(X, W_proj, W3, al3, ar3)
    el12 = el12.reshape(NL, N)
    er12 = er12.reshape(NL, N)
    zf = Z.reshape(2 * NL * N, HW)
    num, den = _stage2(zf, el12, er12, edges_fg.reshape(2 * E),
                       edges_bg.reshape(2 * E), edges_gg.reshape(2 * E))
    den = den.reshape(NL, NC, DR * 16)[:, :, :N].reshape(NL, NC, GRID, 1, R)
    out = _stage3(num, den, h0, query, Wq, Wk,
                  W_out.reshape(1, EMB), b_out.reshape(1, 1))
    return out.reshape(N, 1)
